# baseline (speedup 1.0000x reference)
import functools

import jax
import jax.numpy as jnp
from jax import lax
from jax.experimental import pallas as pl
from jax.experimental.pallas import tpu as pltpu
from jax.experimental.pallas import tpu_sc as plsc

VOCAB = 100000
D = 64
B = 1024
L = 200
NF = 8
NTOT = B * L * NF
NHALF = NTOT // 2
NW = 32
PER_W = NHALF // NW
CH = 128
NCH = PER_W // CH
NB = 8
NG = NCH // NB

_mesh = plsc.VectorSubcoreMesh(core_axis_name="c", subcore_axis_name="s")


@functools.partial(
    pl.kernel,
    out_type=jax.ShapeDtypeStruct((NHALF, D), jnp.float32),
    mesh=_mesh,
    compiler_params=pltpu.CompilerParams(use_tc_tiling_on_sc=False),
    scratch_types=[
        pltpu.VMEM((PER_W,), jnp.int32),
        pltpu.VMEM((NB, CH, D), jnp.float32),
        pltpu.SemaphoreType.DMA((NB,)),
        pltpu.SemaphoreType.DMA((NB,)),
    ],
)
def _emb_gather(idx_hbm, tab_hbm, out_hbm, idx_v, rows_v, gsem, wsem):
    c = lax.axis_index("c")
    s = lax.axis_index("s")
    wid = s * 2 + c
    base_ch = wid * NCH

    pltpu.sync_copy(idx_hbm.at[pl.ds(wid * PER_W, PER_W)], idx_v)

    for b in range(NB):
        pltpu.async_copy(tab_hbm.at[idx_v.at[pl.ds(b * CH, CH)]], rows_v.at[b], gsem.at[b])

    def group_body(g, _):
        for b in range(NB):
            ch = g * NB + b
            pltpu.make_async_copy(
                tab_hbm.at[pl.ds(0, CH)], rows_v.at[b], gsem.at[b]
            ).wait()
            pltpu.async_copy(
                rows_v.at[b], out_hbm.at[pl.ds((base_ch + ch) * CH, CH)],
                wsem.at[b],
            )
        for b in range(NB):
            pltpu.make_async_copy(
                rows_v.at[b], out_hbm.at[pl.ds(0, CH)], wsem.at[b]
            ).wait()

            @pl.when(g + 1 < NG)
            def _():
                nxt = (g + 1) * NB + b
                pltpu.async_copy(
                    tab_hbm.at[idx_v.at[pl.ds(nxt * CH, CH)]], rows_v.at[b],
                    gsem.at[b],
                )

        return 0

    lax.fori_loop(0, NG, group_body, 0)


def kernel(x, tables):
    tab = tables.reshape(NF * VOCAB, D)
    off = (lax.iota(jnp.int32, NTOT) % NF) * VOCAB
    idx = x.reshape(NTOT) + off
    out_a = _emb_gather(idx[:NHALF], tab)
    out_b = _emb_gather(idx[NHALF:], tab)
    out = jnp.concatenate([out_a, out_b], axis=0)
    return out.reshape(B, L, NF * D)


# speedup vs baseline: 2.0893x; 2.0893x over previous
import functools

import jax
import jax.numpy as jnp
from jax import lax
from jax.experimental import pallas as pl
from jax.experimental.pallas import tpu as pltpu
from jax.experimental.pallas import tpu_sc as plsc

VOCAB = 100000
D = 64
B = 1024
L = 200
NF = 8
NTOT = B * L * NF
NW = 32
PER_W = NTOT // NW
CH = 128
NCH = PER_W // CH
NB = 8
NG = NCH // NB

_mesh = plsc.VectorSubcoreMesh(core_axis_name="c", subcore_axis_name="s")


@functools.partial(
    pl.kernel,
    out_type=jax.ShapeDtypeStruct((NTOT, D), jnp.float32),
    mesh=_mesh,
    compiler_params=pltpu.CompilerParams(use_tc_tiling_on_sc=False),
    scratch_types=[
        pltpu.VMEM((PER_W,), jnp.int32),
        pltpu.VMEM((NB, CH, D), jnp.float32),
        pltpu.SemaphoreType.DMA((NB,)),
        pltpu.SemaphoreType.DMA((NB,)),
    ],
)
def _emb_gather(idx_hbm, tab_hbm, out_hbm, idx_v, rows_v, gsem, wsem):
    c = lax.axis_index("c")
    s = lax.axis_index("s")
    wid = s * 2 + c
    base_ch = wid * NCH

    pltpu.sync_copy(idx_hbm.at[pl.ds(wid * PER_W, PER_W)], idx_v)

    for b in range(NB):
        pltpu.async_copy(tab_hbm.at[idx_v.at[pl.ds(b * CH, CH)]], rows_v.at[b], gsem.at[b])

    def group_body(g, _):
        for b in range(NB):
            ch = g * NB + b
            pltpu.make_async_copy(
                tab_hbm.at[pl.ds(0, CH)], rows_v.at[b], gsem.at[b]
            ).wait()
            pltpu.async_copy(
                rows_v.at[b], out_hbm.at[pl.ds((base_ch + ch) * CH, CH)],
                wsem.at[b],
            )
        for b in range(NB):
            pltpu.make_async_copy(
                rows_v.at[b], out_hbm.at[pl.ds(0, CH)], wsem.at[b]
            ).wait()

            @pl.when(g + 1 < NG)
            def _():
                nxt = (g + 1) * NB + b
                pltpu.async_copy(
                    tab_hbm.at[idx_v.at[pl.ds(nxt * CH, CH)]], rows_v.at[b],
                    gsem.at[b],
                )

        return 0

    lax.fori_loop(0, NG, group_body, 0)


def kernel(x, tables):
    tab = tables.reshape(NF * VOCAB, D)
    off = (lax.iota(jnp.int32, NTOT) % NF) * VOCAB
    idx = x.reshape(NTOT) + off
    out = _emb_gather(idx, tab)
    return out.reshape(B, L, NF * D)
